# baseline (speedup 1.0000x reference)
import functools

import jax
import jax.numpy as jnp
from jax import lax
from jax.experimental import pallas as pl
from jax.experimental.pallas import tpu as pltpu
from jax.experimental.pallas import tpu_sc as plsc

T = 1000
B = 16384

_info = plsc.get_sparse_core_info()
NC, NS, L = _info.num_cores, _info.num_subcores, _info.num_lanes
NW = NC * NS
BPW = B // NW


@functools.partial(
    pl.kernel,
    mesh=plsc.VectorSubcoreMesh(core_axis_name="c", subcore_axis_name="s"),
    compiler_params=pltpu.CompilerParams(needs_layout_passes=False),
    out_type=(
        jax.ShapeDtypeStruct((B,), jnp.float32),
        jax.ShapeDtypeStruct((B,), jnp.float32),
        jax.ShapeDtypeStruct((B,), jnp.float32),
    ),
    scratch_types=[
        pltpu.VMEM((T,), jnp.float32),
        pltpu.VMEM((T,), jnp.float32),
        pltpu.VMEM((T,), jnp.float32),
        pltpu.VMEM((BPW,), jnp.int32),
        pltpu.VMEM((BPW,), jnp.float32),
        pltpu.VMEM((BPW,), jnp.float32),
        pltpu.VMEM((BPW,), jnp.float32),
    ],
)
def _gather3(a_hbm, ab_hbm, abp_hbm, idx_hbm,
             oa_hbm, oab_hbm, oabp_hbm,
             ta, tab, tabp, idxv, oa, oab, oabp):
    wid = lax.axis_index("s") * NC + lax.axis_index("c")
    base = wid * BPW
    pltpu.sync_copy(a_hbm, ta)
    pltpu.sync_copy(ab_hbm, tab)
    pltpu.sync_copy(abp_hbm, tabp)
    pltpu.sync_copy(idx_hbm.at[pl.ds(base, BPW)], idxv)
    for i in range(BPW // L):
        sl = pl.ds(i * L, L)
        ix = idxv[sl]
        oa[sl] = plsc.load_gather(ta, [ix])
        oab[sl] = plsc.load_gather(tab, [ix])
        oabp[sl] = plsc.load_gather(tabp, [ix])
    pltpu.sync_copy(oa, oa_hbm.at[pl.ds(base, BPW)])
    pltpu.sync_copy(oab, oab_hbm.at[pl.ds(base, BPW)])
    pltpu.sync_copy(oabp, oabp_hbm.at[pl.ds(base, BPW)])


def kernel(alphas, alpha_bars, alpha_bars_prev, diffusion_steps):
    oa, oab, oabp = _gather3(alphas, alpha_bars, alpha_bars_prev,
                             diffusion_steps)
    shape = (B, 1, 1, 1)
    return oa.reshape(shape), oab.reshape(shape), oabp.reshape(shape)


# speedup vs baseline: 1.0567x; 1.0567x over previous
import functools

import jax
import jax.numpy as jnp
from jax import lax
from jax.experimental import pallas as pl
from jax.experimental.pallas import tpu as pltpu
from jax.experimental.pallas import tpu_sc as plsc

T = 1000
B = 16384

_info = plsc.get_sparse_core_info()
NC, NS, L = _info.num_cores, _info.num_subcores, _info.num_lanes
NW = NC * NS
BPW = B // NW


@functools.partial(
    pl.kernel,
    mesh=plsc.VectorSubcoreMesh(core_axis_name="c", subcore_axis_name="s"),
    compiler_params=pltpu.CompilerParams(needs_layout_passes=False),
    out_type=(
        jax.ShapeDtypeStruct((B,), jnp.float32),
        jax.ShapeDtypeStruct((B,), jnp.float32),
        jax.ShapeDtypeStruct((B,), jnp.float32),
    ),
    scratch_types=[
        pltpu.VMEM((3 * T,), jnp.float32),
        pltpu.VMEM((BPW,), jnp.int32),
        pltpu.VMEM((BPW,), jnp.float32),
        pltpu.VMEM((BPW,), jnp.float32),
        pltpu.VMEM((BPW,), jnp.float32),
    ],
)
def _gather3(tbl_hbm, idx_hbm, oa_hbm, oab_hbm, oabp_hbm,
             tbl, idxv, oa, oab, oabp):
    wid = lax.axis_index("s") * NC + lax.axis_index("c")
    base = wid * BPW
    pltpu.sync_copy(tbl_hbm, tbl)
    pltpu.sync_copy(idx_hbm.at[pl.ds(base, BPW)], idxv)
    for i in range(BPW // L):
        sl = pl.ds(i * L, L)
        ix = idxv[sl]
        oa[sl] = plsc.load_gather(tbl, [ix])
        oab[sl] = plsc.load_gather(tbl, [ix + T])
        oabp[sl] = plsc.load_gather(tbl, [ix + 2 * T])
    pltpu.sync_copy(oa, oa_hbm.at[pl.ds(base, BPW)])
    pltpu.sync_copy(oab, oab_hbm.at[pl.ds(base, BPW)])
    pltpu.sync_copy(oabp, oabp_hbm.at[pl.ds(base, BPW)])


def kernel(alphas, alpha_bars, alpha_bars_prev, diffusion_steps):
    tbl = jnp.concatenate([alphas, alpha_bars, alpha_bars_prev])
    oa, oab, oabp = _gather3(tbl, diffusion_steps)
    shape = (B, 1, 1, 1)
    return oa.reshape(shape), oab.reshape(shape), oabp.reshape(shape)


# speedup vs baseline: 1.0826x; 1.0246x over previous
import functools

import jax
import jax.numpy as jnp
from jax import lax
from jax.experimental import pallas as pl
from jax.experimental.pallas import tpu as pltpu
from jax.experimental.pallas import tpu_sc as plsc

T = 1000
B = 16384

_info = plsc.get_sparse_core_info()
NC, NS, L = _info.num_cores, _info.num_subcores, _info.num_lanes
NW = NC * NS
BPW = B // NW


@functools.partial(
    pl.kernel,
    mesh=plsc.VectorSubcoreMesh(core_axis_name="c", subcore_axis_name="s"),
    compiler_params=pltpu.CompilerParams(needs_layout_passes=False),
    out_type=(
        jax.ShapeDtypeStruct((B,), jnp.float32),
        jax.ShapeDtypeStruct((B,), jnp.float32),
        jax.ShapeDtypeStruct((B,), jnp.float32),
    ),
    scratch_types=[
        pltpu.VMEM((3 * T,), jnp.float32),
        pltpu.VMEM((BPW,), jnp.int32),
        pltpu.VMEM((BPW,), jnp.float32),
        pltpu.VMEM((BPW,), jnp.float32),
        pltpu.VMEM((BPW,), jnp.float32),
        pltpu.SemaphoreType.DMA,
        pltpu.SemaphoreType.DMA,
    ],
)
def _gather3(tbl_hbm, idx_hbm, oa_hbm, oab_hbm, oabp_hbm,
             tbl, idxv, oa, oab, oabp, sem_in, sem_out):
    wid = lax.axis_index("s") * NC + lax.axis_index("c")
    base = wid * BPW
    cp_tbl = pltpu.async_copy(tbl_hbm, tbl, sem_in)
    cp_idx = pltpu.async_copy(idx_hbm.at[pl.ds(base, BPW)], idxv, sem_in)
    cp_idx.wait()
    cp_tbl.wait()
    for i in range(BPW // L):
        sl = pl.ds(i * L, L)
        ix = idxv[sl]
        oa[sl] = plsc.load_gather(tbl, [ix])
        oab[sl] = plsc.load_gather(tbl, [ix + T])
        oabp[sl] = plsc.load_gather(tbl, [ix + 2 * T])
    cp_a = pltpu.async_copy(oa, oa_hbm.at[pl.ds(base, BPW)], sem_out)
    cp_b = pltpu.async_copy(oab, oab_hbm.at[pl.ds(base, BPW)], sem_out)
    cp_c = pltpu.async_copy(oabp, oabp_hbm.at[pl.ds(base, BPW)], sem_out)
    cp_a.wait()
    cp_b.wait()
    cp_c.wait()


def kernel(alphas, alpha_bars, alpha_bars_prev, diffusion_steps):
    tbl = jnp.concatenate([alphas, alpha_bars, alpha_bars_prev])
    oa, oab, oabp = _gather3(tbl, diffusion_steps)
    shape = (B, 1, 1, 1)
    return oa.reshape(shape), oab.reshape(shape), oabp.reshape(shape)
